# baseline (speedup 1.0000x reference)
import jax
import jax.numpy as jnp
from jax.experimental import pallas as pl
from jax.experimental.pallas import tpu as pltpu

B, T, D, H, W = 4, 16, 192, 56, 56
HW = H * W
N = B * T
NBUF = 8


def _body(t_idx_ref, x_ref, pe_ref, o_ref, in_bufs, out_bufs, in_sems, out_sems):
    def in_copy(i, slot):
        return pltpu.make_async_copy(
            x_ref.at[pl.ds(i * D, D), :], in_bufs.at[slot], in_sems.at[slot]
        )

    def out_copy(i, slot):
        return pltpu.make_async_copy(
            out_bufs.at[slot], o_ref.at[pl.ds(i * D, D), :], out_sems.at[slot]
        )

    for i in range(NBUF):
        in_copy(i, i).start()

    for i in range(N):
        slot = i % NBUF
        idx = t_idx_ref[i % T]
        in_copy(i, slot).wait()
        if i >= NBUF:
            out_copy(i - NBUF, slot).wait()
        pev = pe_ref[pl.ds(idx, 1), :]
        out_bufs[slot] = in_bufs[slot] + pev[0][:, None]
        out_copy(i, slot).start()
        if i + NBUF < N:
            in_copy(i + NBUF, slot).start()

    for i in range(N - NBUF, N):
        out_copy(i, i % NBUF).wait()


def kernel(x, t_idx, pe):
    xr = x.reshape(N * D, HW)
    out = pl.pallas_call(
        _body,
        in_specs=[
            pl.BlockSpec(memory_space=pltpu.SMEM),
            pl.BlockSpec(memory_space=pl.ANY),
            pl.BlockSpec(memory_space=pltpu.VMEM),
        ],
        out_specs=pl.BlockSpec(memory_space=pl.ANY),
        out_shape=jax.ShapeDtypeStruct(xr.shape, xr.dtype),
        scratch_shapes=[
            pltpu.VMEM((NBUF, D, HW), jnp.float32),
            pltpu.VMEM((NBUF, D, HW), jnp.float32),
            pltpu.SemaphoreType.DMA((NBUF,)),
            pltpu.SemaphoreType.DMA((NBUF,)),
        ],
    )(t_idx.astype(jnp.int32), xr, pe.reshape(1000, D))
    return out.reshape(B, T, D, H, W)


# speedup vs baseline: 2.5713x; 2.5713x over previous
import jax
import jax.numpy as jnp
from jax.experimental import pallas as pl
from jax.experimental.pallas import tpu as pltpu

B, T, D, H, W = 4, 16, 192, 56, 56
HW = H * W
N = B * T
NBUF = 8


def _body(t_idx_ref, x_ref, pe_ref, o_ref, in_bufs, out_bufs, in_sems, out_sems):
    def in_copy(i, slot):
        return pltpu.make_async_copy(
            x_ref.at[i // T, i % T], in_bufs.at[slot], in_sems.at[slot]
        )

    def out_copy(i, slot):
        return pltpu.make_async_copy(
            out_bufs.at[slot], o_ref.at[i // T, i % T], out_sems.at[slot]
        )

    for i in range(NBUF):
        in_copy(i, i).start()

    for i in range(N):
        slot = i % NBUF
        idx = t_idx_ref[i % T]
        in_copy(i, slot).wait()
        if i >= NBUF:
            out_copy(i - NBUF, slot).wait()
        pev = pe_ref[pl.ds(idx, 1), :]
        out_bufs[slot] = in_bufs[slot] + pev[0][:, None]
        out_copy(i, slot).start()
        if i + NBUF < N:
            in_copy(i + NBUF, slot).start()

    for i in range(N - NBUF, N):
        out_copy(i, i % NBUF).wait()


def kernel(x, t_idx, pe):
    xr = x.reshape(B, T, D, HW)
    out = pl.pallas_call(
        _body,
        in_specs=[
            pl.BlockSpec(memory_space=pltpu.SMEM),
            pl.BlockSpec(memory_space=pl.ANY),
            pl.BlockSpec(memory_space=pltpu.VMEM),
        ],
        out_specs=pl.BlockSpec(memory_space=pl.ANY),
        out_shape=jax.ShapeDtypeStruct(xr.shape, xr.dtype),
        scratch_shapes=[
            pltpu.VMEM((NBUF, D, HW), jnp.float32),
            pltpu.VMEM((NBUF, D, HW), jnp.float32),
            pltpu.SemaphoreType.DMA((NBUF,)),
            pltpu.SemaphoreType.DMA((NBUF,)),
        ],
    )(t_idx.astype(jnp.int32), xr, pe.reshape(1000, D))
    return out.reshape(B, T, D, H, W)


# speedup vs baseline: 2.5838x; 1.0049x over previous
import jax
import jax.numpy as jnp
from jax.experimental import pallas as pl
from jax.experimental.pallas import tpu as pltpu

B, T, D, H, W = 4, 16, 192, 56, 56
HW = H * W
N = B * T
NBUF = 8


def _body(t_idx_ref, x_ref, pe_ref, o_ref, in_bufs, out_bufs, in_sems, out_sems):
    def in_copy(i, slot):
        return pltpu.make_async_copy(
            x_ref.at[i // T, i % T], in_bufs.at[slot], in_sems.at[slot]
        )

    def out_copy(i, slot):
        return pltpu.make_async_copy(
            out_bufs.at[slot], o_ref.at[i // T, i % T], out_sems.at[slot]
        )

    for i in range(NBUF):
        in_copy(i, i).start(priority=i % 2)

    for i in range(N):
        slot = i % NBUF
        idx = t_idx_ref[i % T]
        in_copy(i, slot).wait()
        if i >= NBUF:
            out_copy(i - NBUF, slot).wait()
        pev = pe_ref[pl.ds(idx, 1), :]
        out_bufs[slot] = in_bufs[slot] + pev[0][:, None]
        out_copy(i, slot).start(priority=i % 2)
        if i + NBUF < N:
            in_copy(i + NBUF, slot).start(priority=i % 2)

    for i in range(N - NBUF, N):
        out_copy(i, i % NBUF).wait()


def kernel(x, t_idx, pe):
    xr = x.reshape(B, T, D, HW)
    out = pl.pallas_call(
        _body,
        in_specs=[
            pl.BlockSpec(memory_space=pltpu.SMEM),
            pl.BlockSpec(memory_space=pl.ANY),
            pl.BlockSpec(memory_space=pltpu.VMEM),
        ],
        out_specs=pl.BlockSpec(memory_space=pl.ANY),
        out_shape=jax.ShapeDtypeStruct(xr.shape, xr.dtype),
        scratch_shapes=[
            pltpu.VMEM((NBUF, D, HW), jnp.float32),
            pltpu.VMEM((NBUF, D, HW), jnp.float32),
            pltpu.SemaphoreType.DMA((NBUF,)),
            pltpu.SemaphoreType.DMA((NBUF,)),
        ],
    )(t_idx.astype(jnp.int32), xr, pe.reshape(1000, D))
    return out.reshape(B, T, D, H, W)
